# baseline (speedup 1.0000x reference)
import functools

import jax
import jax.numpy as jnp
from jax import lax
from jax.experimental import pallas as pl
from jax.experimental.pallas import tpu as pltpu
from jax.experimental.pallas import tpu_sc as plsc

_N = 10000
_D = 128
_E = 320000
_NC = 2
_NS = 16
_NW = _NC * _NS
_CH = 125
_NCH = 80
_RPS = _N // _NS


def _sc_mesh():
    return plsc.VectorSubcoreMesh(
        core_axis_name="c", subcore_axis_name="s",
        num_cores=_NC, num_subcores=_NS)


_TP = 10240
_RPP = _TP // _NS


def _sc_degrees(edges):
    @functools.partial(
        pl.kernel,
        out_type=jax.ShapeDtypeStruct((_NC, 2, _TP), jnp.float32),
        mesh=_sc_mesh(),
        scratch_types=[
            pltpu.VMEM((_NCH, _CH), jnp.int32),
            pltpu.VMEM((_NCH, _CH), jnp.int32),
            pltpu.VMEM((128,), jnp.float32),
            pltpu.VMEM((_RPP,), jnp.float32),
            pltpu.VMEM_SHARED((_TP,), jnp.float32),
            pltpu.VMEM_SHARED((_TP,), jnp.float32),
            pltpu.SemaphoreType.DMA,
        ],
    )
    def deg_kernel(edges_hbm, out_hbm, src_v, dst_v, ones_v, zeros_v,
                   acc_s, acc_d, ssem):
        c = lax.axis_index("c")
        s = lax.axis_index("s")
        wid = c * _NS + s

        def fill_zeros(i, _):
            zeros_v[pl.ds(i * 16, 16)] = jnp.zeros((16,), jnp.float32)
            return 0

        lax.fori_loop(0, _RPP // 16, fill_zeros, 0)

        def fill_ones(i, _):
            ones_v[pl.ds(i * 16, 16)] = jnp.ones((16,), jnp.float32)
            return 0

        lax.fori_loop(0, 8, fill_ones, 0)

        pltpu.sync_copy(zeros_v, acc_s.at[pl.ds(s * _RPP, _RPP)])
        pltpu.sync_copy(zeros_v, acc_d.at[pl.ds(s * _RPP, _RPP)])
        pltpu.sync_copy(edges_hbm.at[0, wid], src_v)
        pltpu.sync_copy(edges_hbm.at[1, wid], dst_v)
        plsc.subcore_barrier()

        ones_c = ones_v.at[pl.ds(0, _CH)]

        def body(m, _):
            for u in range(4):
                j = m * 4 + u
                pltpu.async_copy(ones_c, acc_s.at[src_v.at[j]], ssem,
                                 add=True)
                pltpu.async_copy(ones_c, acc_d.at[dst_v.at[j]], ssem,
                                 add=True)
            for u in range(8):
                pltpu.make_async_copy(ones_c, acc_s.at[src_v.at[0]],
                                      ssem).wait()
            return 0

        lax.fori_loop(0, _NCH // 4, body, 0)
        plsc.subcore_barrier()
        pltpu.sync_copy(acc_s.at[pl.ds(s * _RPP, _RPP)],
                        out_hbm.at[c, 0, pl.ds(s * _RPP, _RPP)])
        pltpu.sync_copy(acc_d.at[pl.ds(s * _RPP, _RPP)],
                        out_hbm.at[c, 1, pl.ds(s * _RPP, _RPP)])

    return deg_kernel(edges)


def _sc_edge_agg(table, edges, zeros):
    hch = _NCH // 2

    @functools.partial(
        pl.kernel,
        out_type=jax.ShapeDtypeStruct((_NC, _TP, _D), jnp.float32),
        mesh=_sc_mesh(),
        scratch_types=[
            pltpu.VMEM((_NCH // 2, _CH), jnp.int32),
            pltpu.VMEM((_NCH // 2, _CH), jnp.int32),
            pltpu.VMEM((128, _D), jnp.float32),
            pltpu.VMEM((128, _D), jnp.float32),
            pltpu.VMEM_SHARED((_TP, _D), jnp.float32),
            pltpu.SemaphoreType.DMA,
            pltpu.SemaphoreType.DMA,
        ],
    )
    def agg_kernel(tab_hbm, edges_hbm, zeros_hbm, out_hbm, src_v, dst_v,
                   r0, r1, acc, gsem, ssem):
        c = lax.axis_index("c")
        s = lax.axis_index("s")
        wid = c * _NS + s

        with jax.named_scope("agg_clear"):
            for k in range(_RPP // 128):
                pltpu.async_copy(zeros_hbm,
                                 acc.at[pl.ds(s * _RPP + k * 128, 128)],
                                 gsem)
            for k in range(_RPP // 128):
                pltpu.make_async_copy(
                    zeros_hbm, acc.at[pl.ds(s * _RPP, 128)], gsem).wait()
            plsc.subcore_barrier()

        def wait_gather(buf):
            pltpu.make_async_copy(tab_hbm.at[src_v.at[0]],
                                  buf.at[pl.ds(0, _CH)], gsem).wait()

        for h in range(2):
          with jax.named_scope(f"agg_half{h}"):
            pltpu.sync_copy(edges_hbm.at[0, wid, pl.ds(h * hch, hch)], src_v)
            pltpu.sync_copy(edges_hbm.at[1, wid, pl.ds(h * hch, hch)], dst_v)
            pltpu.async_copy(tab_hbm.at[src_v.at[0]], r0.at[pl.ds(0, _CH)],
                             gsem)
            pltpu.async_copy(tab_hbm.at[src_v.at[1]], r1.at[pl.ds(0, _CH)],
                             gsem)

            def body(m, _):
                j0 = m * 2
                wait_gather(r0)
                pltpu.sync_copy(r0.at[pl.ds(0, _CH)], acc.at[dst_v.at[j0]],
                                add=True)

                @pl.when(m < hch // 2 - 1)
                def _():
                    pltpu.async_copy(tab_hbm.at[src_v.at[j0 + 2]],
                                     r0.at[pl.ds(0, _CH)], gsem)

                wait_gather(r1)
                pltpu.sync_copy(r1.at[pl.ds(0, _CH)],
                                acc.at[dst_v.at[j0 + 1]], add=True)

                @pl.when(m < hch // 2 - 1)
                def _():
                    pltpu.async_copy(tab_hbm.at[src_v.at[j0 + 3]],
                                     r1.at[pl.ds(0, _CH)], gsem)

                return 0

            lax.fori_loop(0, hch // 2, body, 0)
        with jax.named_scope("agg_out"):
            plsc.subcore_barrier()

            for k in range(_RPP // 128):
                pltpu.async_copy(acc.at[pl.ds(s * _RPP + k * 128, 128)],
                                 out_hbm.at[c, pl.ds(s * _RPP + k * 128, 128)],
                                 gsem)
            for k in range(_RPP // 128):
                pltpu.make_async_copy(
                    acc.at[pl.ds(s * _RPP, 128)],
                    out_hbm.at[c, pl.ds(s * _RPP, 128)], gsem).wait()

    return agg_kernel(table, edges, zeros)


_BLK = 1000


def _tc_pre(xp, W1, degs):
    def body(x_ref, w_ref, deg_ref, y_ref, ns_ref, nd_ref):
        deg = deg_ref[0] + deg_ref[1]
        dsrc, ddst = deg[0], deg[1]
        ns = jnp.where(dsrc > 0, lax.rsqrt(jnp.maximum(dsrc, 1.0)), 0.0)
        nd = jnp.where(ddst > 0, lax.rsqrt(jnp.maximum(ddst, 1.0)), 0.0)
        y = jnp.dot(x_ref[...], w_ref[...],
                    preferred_element_type=jnp.float32)
        y_ref[...] = y * ns[:, None]
        ns_ref[...] = ns[:, None]
        nd_ref[...] = nd[:, None]

    blk = 1024
    return pl.pallas_call(
        body,
        grid=(_TP // blk,),
        in_specs=[
            pl.BlockSpec((blk, _D), lambda i: (i, 0)),
            pl.BlockSpec((_D, _D), lambda i: (0, 0)),
            pl.BlockSpec((_NC, 2, blk), lambda i: (0, 0, i)),
        ],
        out_specs=[
            pl.BlockSpec((blk, _D), lambda i: (i, 0)),
            pl.BlockSpec((blk, 1), lambda i: (i, 0)),
            pl.BlockSpec((blk, 1), lambda i: (i, 0)),
        ],
        out_shape=[
            jax.ShapeDtypeStruct((_N, _D), jnp.float32),
            jax.ShapeDtypeStruct((_N, 1), jnp.float32),
            jax.ShapeDtypeStruct((_N, 1), jnp.float32),
        ],
    )(xp, W1, degs)


def _tc_mid(p1, ns, nd, b1, W2):
    def body(p_ref, ns_ref, nd_ref, b_ref, w_ref, y_ref):
        agg = p_ref[0] + p_ref[1]
        h = jnp.maximum(agg * nd_ref[...] + b_ref[...], 0.0)
        y = jnp.dot(h, w_ref[...], preferred_element_type=jnp.float32)
        y_ref[...] = y * ns_ref[...]

    return pl.pallas_call(
        body,
        grid=(_N // _BLK,),
        in_specs=[
            pl.BlockSpec((_NC, _BLK, _D), lambda i: (0, i, 0)),
            pl.BlockSpec((_BLK, 1), lambda i: (i, 0)),
            pl.BlockSpec((_BLK, 1), lambda i: (i, 0)),
            pl.BlockSpec((1, _D), lambda i: (0, 0)),
            pl.BlockSpec((_D, _D), lambda i: (0, 0)),
        ],
        out_specs=pl.BlockSpec((_BLK, _D), lambda i: (i, 0)),
        out_shape=jax.ShapeDtypeStruct((_N, _D), jnp.float32),
    )(p1, ns, nd, b1, W2)


def _tc_post(p2, nd, b2):
    def body(p_ref, nd_ref, b_ref, o_ref):
        agg = p_ref[0] + p_ref[1]
        o_ref[...] = jnp.maximum(agg * nd_ref[...] + b_ref[...], 0.0)

    blk = 1000
    return pl.pallas_call(
        body,
        grid=(_N // blk,),
        in_specs=[
            pl.BlockSpec((_NC, blk, _D), lambda i: (0, i, 0)),
            pl.BlockSpec((blk, 1), lambda i: (i, 0)),
            pl.BlockSpec((1, _D), lambda i: (0, 0)),
        ],
        out_specs=pl.BlockSpec((blk, _D), lambda i: (i, 0)),
        out_shape=jax.ShapeDtypeStruct((_N, _D), jnp.float32),
    )(p2, nd, b2)


def kernel(inputs, edge_index, W1, b1, W2, b2):
    x = inputs
    edges = edge_index.astype(jnp.int32).reshape(2, _NW, _NCH, _CH)
    b1r = b1.reshape(1, _D)
    b2r = b2.reshape(1, _D)

    zeros = jnp.zeros((128, _D), jnp.float32)
    degs = _sc_degrees(edges)
    y1, ns, nd = _tc_pre(x, W1, degs)
    p1 = _sc_edge_agg(y1, edges, zeros)
    y2 = _tc_mid(p1, ns, nd, b1r, W2)
    p2 = _sc_edge_agg(y2, edges, zeros)
    return _tc_post(p2, nd, b2r)


# speedup vs baseline: 1.0504x; 1.0504x over previous
import functools

import jax
import jax.numpy as jnp
from jax import lax
from jax.experimental import pallas as pl
from jax.experimental.pallas import tpu as pltpu
from jax.experimental.pallas import tpu_sc as plsc

_N = 10000
_D = 128
_E = 320000
_NC = 2
_NS = 16
_NW = _NC * _NS
_CH = 125
_NCH = 80
_RPS = _N // _NS


def _sc_mesh():
    return plsc.VectorSubcoreMesh(
        core_axis_name="c", subcore_axis_name="s",
        num_cores=_NC, num_subcores=_NS)


_TP = 10240
_RPP = _TP // _NS


def _sc_degrees(edges):
    @functools.partial(
        pl.kernel,
        out_type=jax.ShapeDtypeStruct((_NC, 2, _TP), jnp.float32),
        mesh=_sc_mesh(),
        scratch_types=[
            pltpu.VMEM((_NCH, _CH), jnp.int32),
            pltpu.VMEM((_NCH, _CH), jnp.int32),
            pltpu.VMEM((128,), jnp.float32),
            pltpu.VMEM((_RPP,), jnp.float32),
            pltpu.VMEM_SHARED((_TP,), jnp.float32),
            pltpu.VMEM_SHARED((_TP,), jnp.float32),
            pltpu.SemaphoreType.DMA,
        ],
    )
    def deg_kernel(edges_hbm, out_hbm, src_v, dst_v, ones_v, zeros_v,
                   acc_s, acc_d, ssem):
        c = lax.axis_index("c")
        s = lax.axis_index("s")
        wid = c * _NS + s

        def fill_zeros(i, _):
            zeros_v[pl.ds(i * 16, 16)] = jnp.zeros((16,), jnp.float32)
            return 0

        lax.fori_loop(0, _RPP // 16, fill_zeros, 0)

        def fill_ones(i, _):
            ones_v[pl.ds(i * 16, 16)] = jnp.ones((16,), jnp.float32)
            return 0

        lax.fori_loop(0, 8, fill_ones, 0)

        pltpu.sync_copy(zeros_v, acc_s.at[pl.ds(s * _RPP, _RPP)])
        pltpu.sync_copy(zeros_v, acc_d.at[pl.ds(s * _RPP, _RPP)])
        pltpu.sync_copy(edges_hbm.at[0, wid], src_v)
        pltpu.sync_copy(edges_hbm.at[1, wid], dst_v)
        plsc.subcore_barrier()

        ones_c = ones_v.at[pl.ds(0, _CH)]

        def body(m, _):
            for u in range(4):
                j = m * 4 + u
                pltpu.async_copy(ones_c, acc_s.at[src_v.at[j]], ssem,
                                 add=True)
                pltpu.async_copy(ones_c, acc_d.at[dst_v.at[j]], ssem,
                                 add=True)
            for u in range(8):
                pltpu.make_async_copy(ones_c, acc_s.at[src_v.at[0]],
                                      ssem).wait()
            return 0

        lax.fori_loop(0, _NCH // 4, body, 0)
        plsc.subcore_barrier()
        pltpu.sync_copy(acc_s.at[pl.ds(s * _RPP, _RPP)],
                        out_hbm.at[c, 0, pl.ds(s * _RPP, _RPP)])
        pltpu.sync_copy(acc_d.at[pl.ds(s * _RPP, _RPP)],
                        out_hbm.at[c, 1, pl.ds(s * _RPP, _RPP)])

    return deg_kernel(edges)


def _sc_edge_agg(table, edges):
    hch = _NCH // 2

    @functools.partial(
        pl.kernel,
        out_type=jax.ShapeDtypeStruct((_NC, _TP, _D), jnp.float32),
        mesh=_sc_mesh(),
        scratch_types=[
            pltpu.VMEM((_NCH // 2, _CH), jnp.int32),
            pltpu.VMEM((_NCH // 2, _CH), jnp.int32),
            pltpu.VMEM((128, _D), jnp.float32),
            pltpu.VMEM((128, _D), jnp.float32),
            pltpu.VMEM_SHARED((_TP, _D), jnp.float32),
            pltpu.SemaphoreType.DMA,
            pltpu.SemaphoreType.DMA,
        ],
    )
    def agg_kernel(tab_hbm, edges_hbm, out_hbm, src_v, dst_v, r0, r1,
                   acc, gsem, ssem):
        c = lax.axis_index("c")
        s = lax.axis_index("s")
        wid = c * _NS + s

        def fill_zeros(i, _):
            r0[i // 8, pl.ds((i % 8) * 16, 16)] = jnp.zeros((16,),
                                                            jnp.float32)
            return 0

        with jax.named_scope("agg_clear"):
            lax.fori_loop(0, 128 * _D // 16, fill_zeros, 0)

            def clear(k, _):
                pltpu.sync_copy(r0, acc.at[pl.ds(s * _RPP + k * 128, 128)])
                return 0

            lax.fori_loop(0, _RPP // 128, clear, 0)
            plsc.subcore_barrier()

        def wait_gather(buf):
            pltpu.make_async_copy(tab_hbm.at[src_v.at[0]],
                                  buf.at[pl.ds(0, _CH)], gsem).wait()

        for h in range(2):
          with jax.named_scope(f"agg_half{h}"):
            pltpu.sync_copy(edges_hbm.at[0, wid, pl.ds(h * hch, hch)], src_v)
            pltpu.sync_copy(edges_hbm.at[1, wid, pl.ds(h * hch, hch)], dst_v)
            pltpu.async_copy(tab_hbm.at[src_v.at[0]], r0.at[pl.ds(0, _CH)],
                             gsem)
            pltpu.async_copy(tab_hbm.at[src_v.at[1]], r1.at[pl.ds(0, _CH)],
                             gsem)

            def body(m, _):
                j0 = m * 2
                wait_gather(r0)
                pltpu.sync_copy(r0.at[pl.ds(0, _CH)], acc.at[dst_v.at[j0]],
                                add=True)

                @pl.when(m < hch // 2 - 1)
                def _():
                    pltpu.async_copy(tab_hbm.at[src_v.at[j0 + 2]],
                                     r0.at[pl.ds(0, _CH)], gsem)

                wait_gather(r1)
                pltpu.sync_copy(r1.at[pl.ds(0, _CH)],
                                acc.at[dst_v.at[j0 + 1]], add=True)

                @pl.when(m < hch // 2 - 1)
                def _():
                    pltpu.async_copy(tab_hbm.at[src_v.at[j0 + 3]],
                                     r1.at[pl.ds(0, _CH)], gsem)

                return 0

            lax.fori_loop(0, hch // 2, body, 0)
        with jax.named_scope("agg_out"):
            plsc.subcore_barrier()

            def copy_out(k, _):
                pltpu.sync_copy(acc.at[pl.ds(s * _RPP + k * 128, 128)],
                                out_hbm.at[c, pl.ds(s * _RPP + k * 128, 128)])
                return 0

            lax.fori_loop(0, _RPP // 128, copy_out, 0)

    return agg_kernel(table, edges)


_BLK = 1000


def _tc_pre(xp, W1, degs):
    def body(x_ref, w_ref, deg_ref, y_ref, ns_ref, nd_ref):
        deg = deg_ref[0] + deg_ref[1]
        dsrc, ddst = deg[0], deg[1]
        ns = jnp.where(dsrc > 0, lax.rsqrt(jnp.maximum(dsrc, 1.0)), 0.0)
        nd = jnp.where(ddst > 0, lax.rsqrt(jnp.maximum(ddst, 1.0)), 0.0)
        y = jnp.dot(x_ref[...], w_ref[...],
                    preferred_element_type=jnp.float32)
        y_ref[...] = y * ns[:, None]
        ns_ref[...] = ns[:, None]
        nd_ref[...] = nd[:, None]

    blk = 1024
    return pl.pallas_call(
        body,
        grid=(_TP // blk,),
        in_specs=[
            pl.BlockSpec((blk, _D), lambda i: (i, 0)),
            pl.BlockSpec((_D, _D), lambda i: (0, 0)),
            pl.BlockSpec((_NC, 2, blk), lambda i: (0, 0, i)),
        ],
        out_specs=[
            pl.BlockSpec((blk, _D), lambda i: (i, 0)),
            pl.BlockSpec((blk, 1), lambda i: (i, 0)),
            pl.BlockSpec((blk, 1), lambda i: (i, 0)),
        ],
        out_shape=[
            jax.ShapeDtypeStruct((_N, _D), jnp.float32),
            jax.ShapeDtypeStruct((_N, 1), jnp.float32),
            jax.ShapeDtypeStruct((_N, 1), jnp.float32),
        ],
    )(xp, W1, degs)


def _tc_mid(p1, ns, nd, b1, W2):
    def body(p_ref, ns_ref, nd_ref, b_ref, w_ref, y_ref):
        agg = p_ref[0] + p_ref[1]
        h = jnp.maximum(agg * nd_ref[...] + b_ref[...], 0.0)
        y = jnp.dot(h, w_ref[...], preferred_element_type=jnp.float32)
        y_ref[...] = y * ns_ref[...]

    return pl.pallas_call(
        body,
        grid=(_N // _BLK,),
        in_specs=[
            pl.BlockSpec((_NC, _BLK, _D), lambda i: (0, i, 0)),
            pl.BlockSpec((_BLK, 1), lambda i: (i, 0)),
            pl.BlockSpec((_BLK, 1), lambda i: (i, 0)),
            pl.BlockSpec((1, _D), lambda i: (0, 0)),
            pl.BlockSpec((_D, _D), lambda i: (0, 0)),
        ],
        out_specs=pl.BlockSpec((_BLK, _D), lambda i: (i, 0)),
        out_shape=jax.ShapeDtypeStruct((_N, _D), jnp.float32),
    )(p1, ns, nd, b1, W2)


def _tc_post(p2, nd, b2):
    def body(p_ref, nd_ref, b_ref, o_ref):
        agg = p_ref[0] + p_ref[1]
        o_ref[...] = jnp.maximum(agg * nd_ref[...] + b_ref[...], 0.0)

    blk = 1000
    return pl.pallas_call(
        body,
        grid=(_N // blk,),
        in_specs=[
            pl.BlockSpec((_NC, blk, _D), lambda i: (0, i, 0)),
            pl.BlockSpec((blk, 1), lambda i: (i, 0)),
            pl.BlockSpec((1, _D), lambda i: (0, 0)),
        ],
        out_specs=pl.BlockSpec((blk, _D), lambda i: (i, 0)),
        out_shape=jax.ShapeDtypeStruct((_N, _D), jnp.float32),
    )(p2, nd, b2)


def kernel(inputs, edge_index, W1, b1, W2, b2):
    x = inputs
    edges = edge_index.astype(jnp.int32).reshape(2, _NW, _NCH, _CH)
    b1r = b1.reshape(1, _D)
    b2r = b2.reshape(1, _D)

    degs = _sc_degrees(edges)
    y1, ns, nd = _tc_pre(x, W1, degs)
    p1 = _sc_edge_agg(y1, edges)
    y2 = _tc_mid(p1, ns, nd, b1r, W2)
    p2 = _sc_edge_agg(y2, edges)
    return _tc_post(p2, nd, b2r)
